# baseline (speedup 1.0000x reference)
import functools

import jax
import jax.numpy as jnp
from jax import lax
from jax.experimental import pallas as pl
from jax.experimental.pallas import tpu as pltpu
from jax.experimental.pallas import tpu_sc as plsc

N = 10000
K = 32
C = 128
H = 128

NC = 2
NS = 16
L = 16
NW = NC * NS
NP = 320
NPAD = NW * NP


def _leaky(x):
    return jnp.where(x >= 0, x, 0.3 * x)


def _tc_pre_body(e_ref, qk_ref, qb_ref, wk1_ref, wb_ref, hid_ref, part_ref):
    e = e_ref[...]
    hid_ref[...] = _leaky(
        jnp.dot(e, qk_ref[...], preferred_element_type=jnp.float32) + qb_ref[...]
    )
    part_ref[...] = (
        jnp.dot(e, wk1_ref[...], preferred_element_type=jnp.float32) + wb_ref[...]
    )


def _tc_pre(e, qk, qb, wk1, wb):
    blk = 1000
    grid = N // blk
    return pl.pallas_call(
        _tc_pre_body,
        grid=(grid,),
        in_specs=[
            pl.BlockSpec((blk, C), lambda i: (i, 0)),
            pl.BlockSpec((C, H), lambda i: (0, 0)),
            pl.BlockSpec((1, H), lambda i: (0, 0)),
            pl.BlockSpec((C, H), lambda i: (0, 0)),
            pl.BlockSpec((1, H), lambda i: (0, 0)),
        ],
        out_specs=[
            pl.BlockSpec((blk, H), lambda i: (i, 0)),
            pl.BlockSpec((blk, H), lambda i: (i, 0)),
        ],
        out_shape=[
            jax.ShapeDtypeStruct((N, H), jnp.float32),
            jax.ShapeDtypeStruct((N, H), jnp.float32),
        ],
    )(e, qk, qb, wk1, wb)


def _tc_post_body(part_ref, ws_ref, wk2_ref, out_ref):
    t = part_ref[...] + jnp.dot(
        ws_ref[...], wk2_ref[...], preferred_element_type=jnp.float32
    )
    h = _leaky(t)
    nrm = jnp.sqrt(jnp.sum(h * h, axis=1, keepdims=True))
    out_ref[...] = h / (nrm + 1e-6)


def _tc_post(part, ws, wk2):
    blk = 1000
    grid = N // blk
    return pl.pallas_call(
        _tc_post_body,
        grid=(grid,),
        in_specs=[
            pl.BlockSpec((blk, H), lambda i: (i, 0)),
            pl.BlockSpec((blk, H), lambda i: (i, 0)),
            pl.BlockSpec((H, H), lambda i: (0, 0)),
        ],
        out_specs=pl.BlockSpec((blk, H), lambda i: (i, 0)),
        out_shape=jax.ShapeDtypeStruct((N, H), jnp.float32),
    )(part, ws, wk2)


def _sc_body(idx_hbm, wflat_hbm, hid_hbm, out_hbm,
             idx_v, widx_v, out_v, hbuf, wbuf, hsem0, hsem1, wsem0, wsem1):
    wid = lax.axis_index("s") * NC + lax.axis_index("c")
    base = wid * NP

    pltpu.sync_copy(idx_hbm.at[pl.ds(base * K, NP * K)], idx_v)

    def widx_body(v, carry):
        e0 = v * L
        lanes = e0 + lax.broadcasted_iota(jnp.int32, (L,), 0)
        node = base + lax.shift_right_logical(lanes, 5)
        node = jnp.minimum(node, N - 1)
        widx_v[pl.ds(e0, L)] = node * N + idx_v[pl.ds(e0, L)]
        return carry

    lax.fori_loop(0, NP * K // L, widx_body, 0)

    hsems = (hsem0, hsem1)
    wsems = (wsem0, wsem1)

    def gathers(i, b):
        h = pltpu.make_async_copy(
            hid_hbm.at[idx_v.at[pl.ds(i * K, K)]], hbuf.at[b], hsems[b]
        )
        w = pltpu.make_async_copy(
            wflat_hbm.at[widx_v.at[pl.ds(i * K, K)]],
            wbuf.at[b, pl.ds(L, K)],
            wsems[b],
        )
        return h, w

    def issue(i, b):
        h, w = gathers(i, b)
        h.start()
        w.start()

    def wait(i, b):
        h, w = gathers(i, b)
        h.wait()
        w.wait()

    lane = lax.broadcasted_iota(jnp.int32, (L,), 0)

    def lane_total(v):
        for s in (8, 4, 2, 1):
            v = v + v.at[lane ^ s].get(mode="promise_in_bounds")
        return v

    def compute(i, b):
        w0 = wbuf[b, pl.ds(L, L)]
        w1 = wbuf[b, pl.ds(2 * L, L)]
        den = lane_total(w0 + w1)
        rb = 1.0 / (den + 1e-6)
        acc = [jnp.zeros((L,), jnp.float32) for _ in range(H // L)]
        for k in range(K):
            wbk = plsc.load_gather(wbuf.at[b], [jnp.full((L,), L + k, jnp.int32)])
            for j in range(H // L):
                acc[j] = acc[j] + wbk * hbuf[b, k, pl.ds(j * L, L)]
        for j in range(H // L):
            out_v[i, pl.ds(j * L, L)] = acc[j] * rb

    issue(0, 0)
    issue(1, 1)

    def step(g, carry):
        i0 = 2 * g
        i1 = i0 + 1
        wait(i0, 0)
        compute(i0, 0)
        issue(jnp.minimum(i0 + 2, NP - 2), 0)
        wait(i1, 1)
        compute(i1, 1)
        issue(jnp.minimum(i1 + 2, NP - 1), 1)
        return carry

    lax.fori_loop(0, NP // 2, step, 0)

    wait(NP - 2, 0)
    wait(NP - 1, 1)

    pltpu.sync_copy(out_v, out_hbm.at[pl.ds(base, NP)])


@functools.partial(
    pl.kernel,
    out_type=jax.ShapeDtypeStruct((NPAD, H), jnp.float32),
    mesh=plsc.VectorSubcoreMesh(
        core_axis_name="c", subcore_axis_name="s", num_cores=NC, num_subcores=NS
    ),
    compiler_params=pltpu.CompilerParams(needs_layout_passes=False),
    scratch_types=[
        pltpu.VMEM((NP * K,), jnp.int32),
        pltpu.VMEM((NP * K,), jnp.int32),
        pltpu.VMEM((NP, H), jnp.float32),
        pltpu.VMEM((2, K, H), jnp.float32),
        pltpu.VMEM((2, L + K), jnp.float32),
        pltpu.SemaphoreType.DMA,
        pltpu.SemaphoreType.DMA,
        pltpu.SemaphoreType.DMA,
        pltpu.SemaphoreType.DMA,
    ],
)
def _sc_aggregate(idx_hbm, wflat_hbm, hid_hbm, out_hbm, *rest):
    _sc_body(idx_hbm, wflat_hbm, hid_hbm, out_hbm, *rest)


def kernel(embeddings, weights, neighbor_set, Qk, Qb, Wk, Wb):
    e = embeddings[0]
    idx = neighbor_set.astype(jnp.int32)
    idx_pad = jnp.zeros((NPAD, K), jnp.int32).at[:N].set(idx).reshape(NPAD * K)
    wflat = weights.reshape(N * N)

    qb2 = Qb.reshape(1, H)
    wb2 = Wb.reshape(1, H)
    wk1 = Wk[:C]
    wk2 = Wk[C:]

    hid, part = _tc_pre(e, Qk, qb2, wk1, wb2)
    ws_pad = _sc_aggregate(idx_pad, wflat, hid)
    out = _tc_post(part, ws_pad[:N], wk2)
    return out[None]
